# baseline (speedup 1.0000x reference)
import functools

import jax
import jax.numpy as jnp
from jax import lax
from jax.experimental import pallas as pl
from jax.experimental.pallas import tpu as pltpu
from jax.experimental.pallas import tpu_sc as plsc

_N_TOK = 16384
_DIM = 2048
_NE = 64
_UTIL = 0.01

_BT = 512
_NW = 32
_TPW = _N_TOK // _NW
_L = 16


def _logits_body(x_ref, wc_ref, eps_ref, out_ref):
    both = lax.dot_general(
        x_ref[...], wc_ref[...], (((1,), (0,)), ((), ())),
        preferred_element_type=jnp.float32)
    g = both[:, :_NE]
    n = both[:, _NE:]
    sp = jnp.maximum(n, 0.0) + jnp.log1p(jnp.exp(-jnp.abs(n)))
    out_ref[...] = g + sp * eps_ref[...]


def _compute_logits(x, wc, eps):
    return pl.pallas_call(
        _logits_body,
        grid=(_N_TOK // _BT,),
        in_specs=[
            pl.BlockSpec((_BT, _DIM), lambda i: (i, 0)),
            pl.BlockSpec((_DIM, 2 * _NE), lambda i: (0, 0)),
            pl.BlockSpec((_BT, _NE), lambda i: (i, 0)),
        ],
        out_specs=pl.BlockSpec((_BT, _NE), lambda i: (i, 0)),
        out_shape=jax.ShapeDtypeStruct((_N_TOK, _NE), jnp.float32),
    )(x, wc, eps)


_BB = 128


def _route(logits_hbm, out_hbm, lg_v, w_v):
    wid = lax.axis_index("s") * 2 + lax.axis_index("c")
    row0 = wid * _TPW

    lanes = lax.iota(jnp.int32, _L)
    neg = jnp.full((_L,), -jnp.inf, jnp.float32)
    zi = jnp.zeros((_L,), jnp.int32)

    def _scan16(tok, e0):
        m1, m2, i1, i2 = neg, neg, zi, zi
        for k in range(_L):
            ev = jnp.full((_L,), e0 + k, jnp.int32)
            v = plsc.load_gather(lg_v, [tok, ev])
            gt1 = v > m1
            gt2 = v > m2
            i2 = jnp.where(gt1, i1, jnp.where(gt2, ev, i2))
            m2 = jnp.where(gt1, m1, jnp.where(gt2, v, m2))
            i1 = jnp.where(gt1, ev, i1)
            m1 = jnp.where(gt1, v, m1)
        return m1, m2, i1, i2

    def _merge(a, b):
        m1a, m2a, i1a, i2a = a
        m1b, m2b, i1b, i2b = b
        afirst = m1a >= m1b
        m1 = jnp.where(afirst, m1a, m1b)
        i1 = jnp.where(afirst, i1a, i1b)
        sa = m2a >= m1b
        sb = m1a >= m2b
        m2 = jnp.where(afirst, jnp.where(sa, m2a, m1b),
                       jnp.where(sb, m1a, m2b))
        i2 = jnp.where(afirst, jnp.where(sa, i2a, i1b),
                       jnp.where(sb, i1a, i2b))
        return m1, m2, i1, i2

    for b in range(_TPW // _BB):
        pltpu.sync_copy(logits_hbm.at[pl.ds(row0 + b * _BB, _BB)], lg_v)

        @plsc.parallel_loop(0, _BB // _L)
        def _group(g):
            tok = g * _L + lanes
            r01 = _merge(_scan16(tok, 0), _scan16(tok, _L))
            r23 = _merge(_scan16(tok, 2 * _L), _scan16(tok, 3 * _L))
            m1, m2, i1, i2 = _merge(r01, r23)
            e2 = jnp.exp(m2 - m1)
            denom = 1.0 + e2
            w1 = 1.0 / denom
            w2 = e2 / denom
            for e in range(_NE):
                ev = jnp.full((_L,), e, jnp.int32)
                val = jnp.where(i1 == ev, w1, jnp.where(i2 == ev, w2, 0.0))
                plsc.store_scatter(w_v, [tok, ev], val)

        pltpu.sync_copy(w_v, out_hbm.at[pl.ds(row0 + b * _BB, _BB)])


@functools.cache
def _route_call():
    mesh = plsc.VectorSubcoreMesh(
        core_axis_name="c", subcore_axis_name="s", num_cores=2,
        num_subcores=16)
    return pl.kernel(
        _route,
        out_type=jax.ShapeDtypeStruct((_N_TOK, _NE), jnp.float32),
        mesh=mesh,
        scratch_types=[
            pltpu.VMEM((_BB, _NE), jnp.float32),
            pltpu.VMEM((_BB, _NE), jnp.float32),
        ],
        compiler_params=pltpu.CompilerParams(needs_layout_passes=False),
    )


def _loss_body(w_ref, out_ref, acc_ref):
    i = pl.program_id(0)

    @pl.when(i == 0)
    def _():
        acc_ref[...] = jnp.zeros_like(acc_ref)

    acc_ref[...] += jnp.sum(w_ref[...], axis=0, keepdims=True)

    @pl.when(i == pl.num_programs(0) - 1)
    def _():
        imp = acc_ref[...]
        mean = jnp.sum(imp) / _NE
        var = jnp.sum((imp - mean) ** 2) / _NE
        out_ref[0, 0] = _UTIL * var / (mean * mean)


def _compute_loss(weights):
    return pl.pallas_call(
        _loss_body,
        grid=(_N_TOK // _BT,),
        in_specs=[pl.BlockSpec((_BT, _NE), lambda i: (i, 0))],
        out_specs=pl.BlockSpec(memory_space=pltpu.SMEM),
        out_shape=jax.ShapeDtypeStruct((1, 1), jnp.float32),
        scratch_shapes=[pltpu.VMEM((1, _NE), jnp.float32)],
    )(weights)


def kernel(x, Wg, Wnoise, noise_eps):
    wc = jnp.concatenate([Wg.T, Wnoise.T], axis=1)
    logits = _compute_logits(x, wc, noise_eps)
    weights = _route_call()(logits)
    loss = _compute_loss(weights)[0, 0]
    return weights, loss


# speedup vs baseline: 1.1784x; 1.1784x over previous
import functools

import jax
import jax.numpy as jnp
from jax import lax
from jax.experimental import pallas as pl
from jax.experimental.pallas import tpu as pltpu
from jax.experimental.pallas import tpu_sc as plsc

_N_TOK = 16384
_DIM = 2048
_NE = 64
_UTIL = 0.01

_BT = 512
_NW = 32
_TPW = _N_TOK // _NW
_L = 16


def _logits_body(x_ref, wc_ref, eps_ref, out_ref):
    both = lax.dot_general(
        x_ref[...], wc_ref[...], (((1,), (0,)), ((), ())),
        preferred_element_type=jnp.float32)
    g = both[:, :_NE]
    n = both[:, _NE:]
    sp = jnp.maximum(n, 0.0) + jnp.log1p(jnp.exp(-jnp.abs(n)))
    out_ref[...] = g + sp * eps_ref[...]


def _compute_logits(x, wc, eps):
    return pl.pallas_call(
        _logits_body,
        grid=(_N_TOK // _BT,),
        in_specs=[
            pl.BlockSpec((_BT, _DIM), lambda i: (i, 0)),
            pl.BlockSpec((_DIM, 2 * _NE), lambda i: (0, 0)),
            pl.BlockSpec((_BT, _NE), lambda i: (i, 0)),
        ],
        out_specs=pl.BlockSpec((_BT, _NE), lambda i: (i, 0)),
        out_shape=jax.ShapeDtypeStruct((_N_TOK, _NE), jnp.float32),
    )(x, wc, eps)


def _route(logits_hbm, i1_hbm, i2_hbm, w1_hbm, w2_hbm,
           lg_v, i1_v, i2_v, w1_v, w2_v):
    wid = lax.axis_index("s") * 2 + lax.axis_index("c")
    row0 = wid * _TPW
    pltpu.sync_copy(logits_hbm.at[pl.ds(row0, _TPW)], lg_v)

    lanes = lax.iota(jnp.int32, _L)
    neg = jnp.full((_L,), -jnp.inf, jnp.float32)
    zi = jnp.zeros((_L,), jnp.int32)

    def _scan16(tok, e0):
        m1, m2, i1, i2 = neg, neg, zi, zi
        for k in range(_L):
            ev = jnp.full((_L,), e0 + k, jnp.int32)
            v = plsc.load_gather(lg_v, [tok, ev])
            gt1 = v > m1
            gt2 = v > m2
            i2 = jnp.where(gt1, i1, jnp.where(gt2, ev, i2))
            m2 = jnp.where(gt1, m1, jnp.where(gt2, v, m2))
            i1 = jnp.where(gt1, ev, i1)
            m1 = jnp.where(gt1, v, m1)
        return m1, m2, i1, i2

    def _merge(a, b):
        m1a, m2a, i1a, i2a = a
        m1b, m2b, i1b, i2b = b
        afirst = m1a >= m1b
        m1 = jnp.where(afirst, m1a, m1b)
        i1 = jnp.where(afirst, i1a, i1b)
        sa = m2a >= m1b
        sb = m1a >= m2b
        m2 = jnp.where(afirst, jnp.where(sa, m2a, m1b),
                       jnp.where(sb, m1a, m2b))
        i2 = jnp.where(afirst, jnp.where(sa, i2a, i1b),
                       jnp.where(sb, i1a, i2b))
        return m1, m2, i1, i2

    @plsc.parallel_loop(0, _TPW // _L)
    def _group(g):
        tok = g * _L + lanes
        r01 = _merge(_scan16(tok, 0), _scan16(tok, _L))
        r23 = _merge(_scan16(tok, 2 * _L), _scan16(tok, 3 * _L))
        m1, m2, i1, i2 = _merge(r01, r23)
        e2 = jnp.exp(m2 - m1)
        denom = 1.0 + e2
        base = g * _L
        i1_v[0, pl.ds(base, _L)] = i1
        i2_v[0, pl.ds(base, _L)] = i2
        w1_v[0, pl.ds(base, _L)] = 1.0 / denom
        w2_v[0, pl.ds(base, _L)] = e2 / denom

    pltpu.sync_copy(i1_v, i1_hbm.at[wid])
    pltpu.sync_copy(i2_v, i2_hbm.at[wid])
    pltpu.sync_copy(w1_v, w1_hbm.at[wid])
    pltpu.sync_copy(w2_v, w2_hbm.at[wid])


@functools.cache
def _route_call():
    mesh = plsc.VectorSubcoreMesh(
        core_axis_name="c", subcore_axis_name="s", num_cores=2,
        num_subcores=16)
    return pl.kernel(
        _route,
        out_type=(
            jax.ShapeDtypeStruct((_NW, 1, _TPW), jnp.int32),
            jax.ShapeDtypeStruct((_NW, 1, _TPW), jnp.int32),
            jax.ShapeDtypeStruct((_NW, 1, _TPW), jnp.float32),
            jax.ShapeDtypeStruct((_NW, 1, _TPW), jnp.float32),
        ),
        mesh=mesh,
        scratch_types=[
            pltpu.VMEM((_TPW, _NE), jnp.float32),
            pltpu.VMEM((1, _TPW), jnp.int32),
            pltpu.VMEM((1, _TPW), jnp.int32),
            pltpu.VMEM((1, _TPW), jnp.float32),
            pltpu.VMEM((1, _TPW), jnp.float32),
        ],
        compiler_params=pltpu.CompilerParams(needs_layout_passes=False),
    )


def _expand_body(i1_ref, i2_ref, w1_ref, w2_ref, w_out_ref, loss_ref, acc_ref):
    i = pl.program_id(0)

    @pl.when(i == 0)
    def _():
        acc_ref[...] = jnp.zeros_like(acc_ref)

    i1 = i1_ref[...].reshape(_TPW, 1)
    i2 = i2_ref[...].reshape(_TPW, 1)
    w1 = w1_ref[...].reshape(_TPW, 1)
    w2 = w2_ref[...].reshape(_TPW, 1)
    cols = lax.broadcasted_iota(jnp.int32, (_TPW, _NE), 1)
    w = (jnp.where(cols == i1, w1, 0.0)
         + jnp.where(cols == i2, w2, 0.0))
    w_out_ref[...] = w
    acc_ref[...] += jnp.sum(w, axis=0, keepdims=True)

    @pl.when(i == pl.num_programs(0) - 1)
    def _():
        imp = acc_ref[...]
        mean = jnp.sum(imp) / _NE
        var = jnp.sum((imp - mean) ** 2) / _NE
        loss_ref[0, 0] = _UTIL * var / (mean * mean)


def _expand(i1, i2, w1, w2):
    spec = lambda: pl.BlockSpec((1, 1, _TPW), lambda i: (i, 0, 0))
    return pl.pallas_call(
        _expand_body,
        grid=(_NW,),
        in_specs=[spec(), spec(), spec(), spec()],
        out_specs=(
            pl.BlockSpec((_TPW, _NE), lambda i: (i, 0)),
            pl.BlockSpec(memory_space=pltpu.SMEM),
        ),
        out_shape=(
            jax.ShapeDtypeStruct((_N_TOK, _NE), jnp.float32),
            jax.ShapeDtypeStruct((1, 1), jnp.float32),
        ),
        scratch_shapes=[pltpu.VMEM((1, _NE), jnp.float32)],
    )(i1, i2, w1, w2)


def kernel(x, Wg, Wnoise, noise_eps):
    wc = jnp.concatenate([Wg.T, Wnoise.T], axis=1)
    logits = _compute_logits(x, wc, noise_eps)
    i1, i2, w1, w2 = _route_call()(logits)
    weights, loss = _expand(i1, i2, w1, w2)
    return weights, loss[0, 0]


# speedup vs baseline: 1.2793x; 1.0856x over previous
import functools

import jax
import jax.numpy as jnp
from jax import lax
from jax.experimental import pallas as pl
from jax.experimental.pallas import tpu as pltpu
from jax.experimental.pallas import tpu_sc as plsc

_N_TOK = 16384
_DIM = 2048
_NE = 64
_UTIL = 0.01

_NH = 2
_HT = _N_TOK // _NH
_BT = 1024
_NW = 32
_TPW = _HT // _NW
_L = 16


def _logits_body(x_ref, wc_ref, eps_ref, out_ref):
    both = lax.dot_general(
        x_ref[...], wc_ref[...], (((1,), (0,)), ((), ())),
        preferred_element_type=jnp.float32)
    g = both[:, :_NE]
    n = both[:, _NE:]
    sp = jnp.maximum(n, 0.0) + jnp.log1p(jnp.exp(-jnp.abs(n)))
    out_ref[...] = g + sp * eps_ref[...]


def _compute_logits(x, wc, eps, h):
    nblk = _HT // _BT
    return pl.pallas_call(
        _logits_body,
        grid=(nblk,),
        in_specs=[
            pl.BlockSpec((_BT, _DIM), lambda i: (i + h * nblk, 0)),
            pl.BlockSpec((_DIM, 2 * _NE), lambda i: (0, 0)),
            pl.BlockSpec((_BT, _NE), lambda i: (i + h * nblk, 0)),
        ],
        out_specs=pl.BlockSpec((_BT, _NE), lambda i: (i, 0)),
        out_shape=jax.ShapeDtypeStruct((_HT, _NE), jnp.float32),
    )(x, wc, eps)


def _route(logits_hbm, i1_hbm, i2_hbm, w1_hbm, w2_hbm,
           lg_v, i1_v, i2_v, w1_v, w2_v):
    wid = lax.axis_index("s") * 2 + lax.axis_index("c")
    row0 = wid * _TPW
    pltpu.sync_copy(logits_hbm.at[pl.ds(row0, _TPW)], lg_v)

    lanes = lax.iota(jnp.int32, _L)
    neg = jnp.full((_L,), -jnp.inf, jnp.float32)
    zi = jnp.zeros((_L,), jnp.int32)

    _NC = 4
    _CL = _NE // _NC

    def _scan_chains(tok):
        st = [[neg, neg, zi, zi] for _ in range(_NC)]
        evs = [[jnp.full((_L,), c * _CL + k, jnp.int32) for c in range(_NC)]
               for k in range(_CL)]
        cur = [plsc.load_gather(lg_v, [tok, evs[0][c]]) for c in range(_NC)]
        for k in range(_CL):
            nxt = ([plsc.load_gather(lg_v, [tok, evs[k + 1][c]])
                    for c in range(_NC)] if k + 1 < _CL else None)
            for c in range(_NC):
                m1, m2, i1, i2 = st[c]
                v = cur[c]
                ev = evs[k][c]
                gt1 = v > m1
                gt2 = v > m2
                st[c] = [
                    jnp.where(gt1, v, m1),
                    jnp.where(gt1, m1, jnp.where(gt2, v, m2)),
                    jnp.where(gt1, ev, i1),
                    jnp.where(gt1, i1, jnp.where(gt2, ev, i2)),
                ]
            cur = nxt
        return [tuple(s) for s in st]

    def _merge(a, b):
        m1a, m2a, i1a, i2a = a
        m1b, m2b, i1b, i2b = b
        afirst = m1a >= m1b
        m1 = jnp.where(afirst, m1a, m1b)
        i1 = jnp.where(afirst, i1a, i1b)
        sa = m2a >= m1b
        sb = m1a >= m2b
        m2 = jnp.where(afirst, jnp.where(sa, m2a, m1b),
                       jnp.where(sb, m1a, m2b))
        i2 = jnp.where(afirst, jnp.where(sa, i2a, i1b),
                       jnp.where(sb, i1a, i2b))
        return m1, m2, i1, i2

    @plsc.parallel_loop(0, _TPW // _L)
    def _group(g):
        tok = g * _L + lanes
        c0, c1, c2, c3 = _scan_chains(tok)
        m1, m2, i1, i2 = _merge(_merge(c0, c1), _merge(c2, c3))
        e2 = jnp.exp(m2 - m1)
        denom = 1.0 + e2
        base = g * _L
        i1_v[0, pl.ds(base, _L)] = i1
        i2_v[0, pl.ds(base, _L)] = i2
        w1_v[0, pl.ds(base, _L)] = 1.0 / denom
        w2_v[0, pl.ds(base, _L)] = e2 / denom

    pltpu.sync_copy(i1_v, i1_hbm.at[wid])
    pltpu.sync_copy(i2_v, i2_hbm.at[wid])
    pltpu.sync_copy(w1_v, w1_hbm.at[wid])
    pltpu.sync_copy(w2_v, w2_hbm.at[wid])


@functools.cache
def _route_call():
    mesh = plsc.VectorSubcoreMesh(
        core_axis_name="c", subcore_axis_name="s", num_cores=2,
        num_subcores=16)
    return pl.kernel(
        _route,
        out_type=(
            jax.ShapeDtypeStruct((_NW, 1, _TPW), jnp.int32),
            jax.ShapeDtypeStruct((_NW, 1, _TPW), jnp.int32),
            jax.ShapeDtypeStruct((_NW, 1, _TPW), jnp.float32),
            jax.ShapeDtypeStruct((_NW, 1, _TPW), jnp.float32),
        ),
        mesh=mesh,
        scratch_types=[
            pltpu.VMEM((_TPW, _NE), jnp.float32),
            pltpu.VMEM((1, _TPW), jnp.int32),
            pltpu.VMEM((1, _TPW), jnp.int32),
            pltpu.VMEM((1, _TPW), jnp.float32),
            pltpu.VMEM((1, _TPW), jnp.float32),
        ],
        compiler_params=pltpu.CompilerParams(needs_layout_passes=False),
    )


def _flat_col(ref):
    parts = [ref[h:h + 1] for h in range(_NH)]
    cat = jnp.concatenate(parts, axis=2)
    return cat.reshape(_NH * _TPW, 1)


def _expand_body(i1_ref, i2_ref, w1_ref, w2_ref, w_out_ref, loss_ref, acc_ref):
    i = pl.program_id(0)

    @pl.when(i == 0)
    def _():
        acc_ref[...] = jnp.zeros_like(acc_ref)

    nrow = _NH * _TPW
    i1 = _flat_col(i1_ref)
    i2 = _flat_col(i2_ref)
    w1 = _flat_col(w1_ref)
    w2 = _flat_col(w2_ref)
    cols = lax.broadcasted_iota(jnp.int32, (nrow, _NE), 1)
    w = (jnp.where(cols == i1, w1, 0.0)
         + jnp.where(cols == i2, w2, 0.0))
    w_out_ref[...] = w
    acc_ref[...] += jnp.sum(w, axis=0, keepdims=True)

    @pl.when(i == pl.num_programs(0) - 1)
    def _():
        imp = acc_ref[...]
        mean = jnp.sum(imp) / _NE
        var = jnp.sum((imp - mean) ** 2) / _NE
        loss_ref[0, 0] = _UTIL * var / (mean * mean)


def _expand(i1, i2, w1, w2):
    spec = lambda: pl.BlockSpec((_NH, 1, _TPW), lambda i: (i, 0, 0))
    return pl.pallas_call(
        _expand_body,
        grid=(_NW,),
        in_specs=[spec(), spec(), spec(), spec()],
        out_specs=(
            pl.BlockSpec((_NH * _TPW, _NE), lambda i: (i, 0)),
            pl.BlockSpec(memory_space=pltpu.SMEM),
        ),
        out_shape=(
            jax.ShapeDtypeStruct((_N_TOK, _NE), jnp.float32),
            jax.ShapeDtypeStruct((1, 1), jnp.float32),
        ),
        scratch_shapes=[pltpu.VMEM((1, _NE), jnp.float32)],
    )(i1, i2, w1, w2)


def kernel(x, Wg, Wnoise, noise_eps):
    wc = jnp.concatenate([Wg.T, Wnoise.T], axis=1)
    route = _route_call()
    outs = []
    for h in range(_NH):
        logits_h = _compute_logits(x, wc, noise_eps, h)
        outs.append(route(logits_h))
    i1, i2, w1, w2 = (jnp.concatenate(parts, axis=0)
                      for parts in zip(*outs))
    weights, loss = _expand(i1, i2, w1, w2)
    return weights, loss[0, 0]


# speedup vs baseline: 1.3840x; 1.0818x over previous
import functools

import jax
import jax.numpy as jnp
from jax import lax
from jax.experimental import pallas as pl
from jax.experimental.pallas import tpu as pltpu
from jax.experimental.pallas import tpu_sc as plsc

_N_TOK = 16384
_DIM = 2048
_NE = 64
_UTIL = 0.01

_NH = 2
_HT = _N_TOK // _NH
_BT = 1024
_NW = 32
_TPW = _HT // _NW
_L = 16


def _logits_body(x_ref, wc_ref, eps_ref, out_ref):
    both = lax.dot_general(
        x_ref[...], wc_ref[...], (((1,), (0,)), ((), ())),
        preferred_element_type=jnp.float32)
    g = both[:, :_NE]
    n = both[:, _NE:]
    sp = jnp.maximum(n, 0.0) + jnp.log1p(jnp.exp(-jnp.abs(n)))
    out_ref[...] = g + sp * eps_ref[...]


def _compute_logits(x, wc, eps, h):
    nblk = _HT // _BT
    return pl.pallas_call(
        _logits_body,
        grid=(nblk,),
        in_specs=[
            pl.BlockSpec((_BT, _DIM), lambda i: (i + h * nblk, 0)),
            pl.BlockSpec((_DIM, 2 * _NE), lambda i: (0, 0)),
            pl.BlockSpec((_BT, _NE), lambda i: (i + h * nblk, 0)),
        ],
        out_specs=pl.BlockSpec((_BT, _NE), lambda i: (i, 0)),
        out_shape=jax.ShapeDtypeStruct((_HT, _NE), jnp.float32),
    )(x, wc, eps)


def _route(logits_hbm, pk_hbm, w1_hbm, lg_v, pk_v, w1_v):
    wid = lax.axis_index("s") * 2 + lax.axis_index("c")
    row0 = wid * _TPW
    pltpu.sync_copy(logits_hbm.at[pl.ds(row0, _TPW)], lg_v)

    lanes = lax.iota(jnp.int32, _L)
    neg = jnp.full((_L,), -jnp.inf, jnp.float32)
    zi = jnp.zeros((_L,), jnp.int32)

    _NC = 4
    _CL = _NE // _NC

    def _scan_chains(tok):
        st = [[neg, neg, zi, zi] for _ in range(_NC)]
        evs = [[jnp.full((_L,), c * _CL + k, jnp.int32) for c in range(_NC)]
               for k in range(_CL)]
        cur = [plsc.load_gather(lg_v, [tok, evs[0][c]]) for c in range(_NC)]
        for k in range(_CL):
            nxt = ([plsc.load_gather(lg_v, [tok, evs[k + 1][c]])
                    for c in range(_NC)] if k + 1 < _CL else None)
            for c in range(_NC):
                m1, m2, i1, i2 = st[c]
                v = cur[c]
                ev = evs[k][c]
                gt1 = v > m1
                gt2 = v > m2
                st[c] = [
                    jnp.where(gt1, v, m1),
                    jnp.where(gt1, m1, jnp.where(gt2, v, m2)),
                    jnp.where(gt1, ev, i1),
                    jnp.where(gt1, i1, jnp.where(gt2, ev, i2)),
                ]
            cur = nxt
        return [tuple(s) for s in st]

    def _merge(a, b):
        m1a, m2a, i1a, i2a = a
        m1b, m2b, i1b, i2b = b
        afirst = m1a >= m1b
        m1 = jnp.where(afirst, m1a, m1b)
        i1 = jnp.where(afirst, i1a, i1b)
        sa = m2a >= m1b
        sb = m1a >= m2b
        m2 = jnp.where(afirst, jnp.where(sa, m2a, m1b),
                       jnp.where(sb, m1a, m2b))
        i2 = jnp.where(afirst, jnp.where(sa, i2a, i1b),
                       jnp.where(sb, i1a, i2b))
        return m1, m2, i1, i2

    @plsc.parallel_loop(0, _TPW // _L)
    def _group(g):
        tok = g * _L + lanes
        c0, c1, c2, c3 = _scan_chains(tok)
        m1, m2, i1, i2 = _merge(_merge(c0, c1), _merge(c2, c3))
        e2 = jnp.exp(m2 - m1)
        base = g * _L
        pk_v[0, pl.ds(base, _L)] = jnp.left_shift(i1, 8) | i2
        w1_v[0, pl.ds(base, _L)] = 1.0 / (1.0 + e2)

    pltpu.sync_copy(pk_v, pk_hbm.at[wid])
    pltpu.sync_copy(w1_v, w1_hbm.at[wid])


@functools.cache
def _route_call():
    mesh = plsc.VectorSubcoreMesh(
        core_axis_name="c", subcore_axis_name="s", num_cores=2,
        num_subcores=16)
    return pl.kernel(
        _route,
        out_type=(
            jax.ShapeDtypeStruct((_NW, 1, _TPW), jnp.int32),
            jax.ShapeDtypeStruct((_NW, 1, _TPW), jnp.float32),
        ),
        mesh=mesh,
        scratch_types=[
            pltpu.VMEM((_TPW, _NE), jnp.float32),
            pltpu.VMEM((1, _TPW), jnp.int32),
            pltpu.VMEM((1, _TPW), jnp.float32),
        ],
        compiler_params=pltpu.CompilerParams(needs_layout_passes=False),
    )


_RB = 4


def _flat_col(ref):
    parts = [ref[h:h + 1] for h in range(_RB)]
    cat = jnp.concatenate(parts, axis=2)
    return cat.reshape(_RB * _TPW, 1)


def _expand_body(pk_ref, w1_ref, w_out_ref, loss_ref, acc_ref):
    i = pl.program_id(0)

    @pl.when(i == 0)
    def _():
        acc_ref[...] = jnp.zeros_like(acc_ref)

    nrow = _RB * _TPW
    pk = _flat_col(pk_ref)
    w1 = _flat_col(w1_ref)
    i1 = lax.shift_right_logical(pk, 8)
    i2 = pk & 0xFF
    w2 = 1.0 - w1
    cols = lax.broadcasted_iota(jnp.int32, (nrow, _NE), 1)
    w = (jnp.where(cols == i1, w1, 0.0)
         + jnp.where(cols == i2, w2, 0.0))
    w_out_ref[...] = w
    acc_ref[...] += jnp.sum(w, axis=0, keepdims=True)

    @pl.when(i == pl.num_programs(0) - 1)
    def _():
        imp = acc_ref[...]
        mean = jnp.sum(imp) / _NE
        var = jnp.sum((imp - mean) ** 2) / _NE
        loss_ref[0, 0] = _UTIL * var / (mean * mean)


def _expand(pk, w1):
    spec = lambda: pl.BlockSpec((_RB, 1, _TPW), lambda i: (i, 0, 0))
    return pl.pallas_call(
        _expand_body,
        grid=(_NH * _NW // _RB,),
        in_specs=[spec(), spec()],
        out_specs=(
            pl.BlockSpec((_RB * _TPW, _NE), lambda i: (i, 0)),
            pl.BlockSpec(memory_space=pltpu.SMEM),
        ),
        out_shape=(
            jax.ShapeDtypeStruct((_N_TOK, _NE), jnp.float32),
            jax.ShapeDtypeStruct((1, 1), jnp.float32),
        ),
        scratch_shapes=[pltpu.VMEM((1, _NE), jnp.float32)],
    )(pk, w1)


def kernel(x, Wg, Wnoise, noise_eps):
    wc = jnp.concatenate([Wg.T, Wnoise.T], axis=1)
    route = _route_call()
    outs = []
    for h in range(_NH):
        logits_h = _compute_logits(x, wc, noise_eps, h)
        outs.append(route(logits_h))
    pk, w1 = (jnp.concatenate(parts, axis=0) for parts in zip(*outs))
    weights, loss = _expand(pk, w1)
    return weights, loss[0, 0]
